# Initial kernel scaffold; baseline (speedup 1.0000x reference)
import jax
import jax.numpy as jnp
from jax.experimental import pallas as pl


def kernel(verts, edge_index, W1_0, b1_0, W2_0, b2_0, W1_1, b1_1, W2_1, b2_1, W1_2, b1_2, W2_2, b2_2, Wo1, bo1, Wo2, bo2):
    raise NotImplementedError("write your pallas kernel here")


# speedup vs baseline: 5.3582x; 5.3582x over previous
import functools

import jax
import jax.numpy as jnp
from jax import lax
from jax.experimental import pallas as pl
from jax.experimental.pallas import tpu as pltpu
from jax.experimental.pallas import tpu_sc as plsc

N = 10000
E = 320000
H = 128
WIN = 128
NWIN = E // WIN
NSC = 2
NSUB = 16
ROWS_A = 624
ROWS_LAST = N - (NSUB - 1) * ROWS_A
HP = H // 2


def _pack_bf16_pair(x):
  u = lax.bitcast_convert_type(x, jnp.uint32)
  r = u + jnp.uint32(0x7FFF) + ((u >> 16) & jnp.uint32(1))
  lo = r[:, :HP] >> 16
  hi = r[:, HP:] & jnp.uint32(0xFFFF0000)
  return lax.bitcast_convert_type(lo | hi, jnp.float32)


def _unpack_bf16_pair(v):
  u = lax.bitcast_convert_type(v, jnp.uint32)
  lo = lax.bitcast_convert_type(u << 16, jnp.float32)
  hi = lax.bitcast_convert_type(u & jnp.uint32(0xFFFF0000), jnp.float32)
  return jnp.concatenate([lo, hi], axis=1)


def _rowwise_copy(s, copy_fn):
  @pl.when(s < NSUB - 1)
  def _():
    copy_fn(pl.multiple_of(s * ROWS_A, 8), ROWS_A)

  @pl.when(s == NSUB - 1)
  def _():
    copy_fn((NSUB - 1) * ROWS_A, ROWS_LAST)

@functools.cache
def _mesh():
  return plsc.VectorSubcoreMesh(core_axis_name="core", subcore_axis_name="subcore")


def _gather_body(p_hbm, q_hbm, i_hbm, j_hbm, gi_hbm, gj_hbm, sem):
  def body(i_vmem, j_vmem, gi_vmem, gj_vmem):
    a = pltpu.async_copy(p_hbm.at[i_vmem.at[0]], gi_vmem, sem)
    b = pltpu.async_copy(q_hbm.at[j_vmem.at[0]], gj_vmem, sem)
    a.wait()
    b.wait()

  pltpu.emit_pipeline(
      body,
      grid=(NWIN,),
      in_specs=[
          pl.BlockSpec((1, WIN), lambda g: (0, g)),
          pl.BlockSpec((1, WIN), lambda g: (0, g)),
      ],
      out_specs=[
          pl.BlockSpec((WIN, H), lambda g: (g, 0)),
          pl.BlockSpec((WIN, H), lambda g: (g, 0)),
      ],
      core_axis_name=("core", "subcore"),
      dimension_semantics=(pltpu.PARALLEL,),
  )(i_hbm, j_hbm, gi_hbm, gj_hbm)


def _sc_gather(p, q, idx_i, idx_j):
  out = jax.ShapeDtypeStruct((E, H), jnp.float32)
  k = pl.kernel(
      _gather_body,
      out_type=(out, out),
      mesh=_mesh(),
      scratch_types=[pltpu.SemaphoreType.DMA],
  )
  return k(p, q, idx_i, idx_j)


def _scatter_body(m_hbm, j_hbm, z_hbm, out_hbm, acc, sem):
  c = lax.axis_index("core")
  s = lax.axis_index("subcore")
  _rowwise_copy(s, lambda r0, nr: pltpu.async_copy(
      z_hbm.at[pl.ds(r0, nr)], acc.at[pl.ds(r0, nr)], sem).wait())
  plsc.subcore_barrier()

  def body(m_vmem, j_vmem):
    pltpu.sync_copy(m_vmem, acc.at[j_vmem.at[0]], add=True)

  pltpu.emit_pipeline(
      body,
      grid=(NWIN,),
      in_specs=[
          pl.BlockSpec((WIN, H), lambda g: (g, 0)),
          pl.BlockSpec((1, WIN), lambda g: (0, g)),
      ],
      out_specs=[],
      core_axis_name=("core", "subcore"),
      dimension_semantics=(pltpu.PARALLEL,),
  )(m_hbm, j_hbm)
  plsc.subcore_barrier()
  _rowwise_copy(s, lambda r0, nr: pltpu.async_copy(
      acc.at[pl.ds(r0, nr)], out_hbm.at[c, pl.ds(r0, nr)], sem).wait())


def _sc_scatter_add(messages, idx_j, zeros_nh):
  k = pl.kernel(
      _scatter_body,
      out_type=jax.ShapeDtypeStruct((NSC, N, H), jnp.float32),
      mesh=_mesh(),
      scratch_types=[
          pltpu.VMEM_SHARED((N, H), jnp.float32),
          pltpu.SemaphoreType.DMA,
      ],
  )
  return k(messages, idx_j, zeros_nh)


def _degree_body(j_hbm, z_hbm, ones_hbm, out_hbm, acc, ones, sem):
  c = lax.axis_index("core")
  s = lax.axis_index("subcore")
  pltpu.async_copy(ones_hbm, ones, sem).wait()
  _rowwise_copy(s, lambda r0, nr: pltpu.async_copy(
      z_hbm.at[pl.ds(r0, nr)], acc.at[pl.ds(r0, nr)], sem).wait())
  plsc.subcore_barrier()

  def body(j_vmem):
    pltpu.sync_copy(ones, acc.at[j_vmem.at[0]], add=True)

  pltpu.emit_pipeline(
      body,
      grid=(NWIN,),
      in_specs=[pl.BlockSpec((1, WIN), lambda g: (0, g))],
      out_specs=[],
      core_axis_name=("core", "subcore"),
      dimension_semantics=(pltpu.PARALLEL,),
  )(j_hbm)
  plsc.subcore_barrier()
  _rowwise_copy(s, lambda r0, nr: pltpu.async_copy(
      acc.at[pl.ds(r0, nr)], out_hbm.at[c, pl.ds(r0, nr)], sem).wait())


def _sc_degree(idx_j, zeros_nh, ones_wh):
  k = pl.kernel(
      _degree_body,
      out_type=jax.ShapeDtypeStruct((NSC, N, H), jnp.float32),
      mesh=_mesh(),
      scratch_types=[
          pltpu.VMEM_SHARED((N, H), jnp.float32),
          pltpu.VMEM((WIN, H), jnp.float32),
          pltpu.SemaphoreType.DMA,
      ],
  )
  return k(idx_j, zeros_nh, ones_wh)


def _pq0_tc(x_ref, w1_ref, b1_ref, p_ref, q_ref):
  x = x_ref[...]
  wa = w1_ref[0:3, :]
  wb = w1_ref[3:6, :]
  p_ref[...] = jnp.dot(x, wa - wb, preferred_element_type=jnp.float32) + b1_ref[...]
  q_ref[...] = jnp.dot(x, wb, preferred_element_type=jnp.float32)


def _pq0(verts, w1, b1):
  out = jax.ShapeDtypeStruct((N, H), jnp.float32)
  return pl.pallas_call(
      _pq0_tc, out_shape=(out, out),
  )(verts, w1, b1.reshape(1, H))


def _combine_pq_tc(s_ref, dp_ref, xres_ref, w1_ref, b1_ref,
                   x_ref, p_ref, q_ref):
  d = dp_ref[0, :, 0:1] + dp_ref[1, :, 0:1]
  rdeg = 1.0 / jnp.maximum(d, 1.0)
  x = (s_ref[0] + s_ref[1]) * rdeg
  if xres_ref is not None:
    x = x + xres_ref[...]
  x_ref[...] = x
  wa = w1_ref[0:H, :]
  wb = w1_ref[H:2 * H, :]
  p_ref[...] = jnp.dot(x, wa - wb, preferred_element_type=jnp.float32) + b1_ref[...]
  q_ref[...] = jnp.dot(x, wb, preferred_element_type=jnp.float32)


def _combine_pq(s_part, deg_part, x_res, w1, b1):
  xout = jax.ShapeDtypeStruct((N, H), jnp.float32)
  out = jax.ShapeDtypeStruct((N, H), jnp.float32)
  if x_res is None:
    body = lambda s, dp, w, b, xo, po, qo: _combine_pq_tc(s, dp, None, w, b, xo, po, qo)
    return pl.pallas_call(body, out_shape=(xout, out, out))(
        s_part, deg_part, w1, b1.reshape(1, H))
  return pl.pallas_call(_combine_pq_tc, out_shape=(xout, out, out))(
      s_part, deg_part, x_res, w1, b1.reshape(1, H))


_MSG_BLK = 2000


def _msg_tc(gi_ref, gj_ref, w2_ref, b2_ref, m_ref):
  h = jnp.maximum(gi_ref[...] + gj_ref[...], 0.0)
  y = jnp.dot(h, w2_ref[...], preferred_element_type=jnp.float32) + b2_ref[...]
  m_ref[...] = jnp.maximum(y, 0.0)


def _messages(gi, gj, w2, b2):
  grid = (E // _MSG_BLK,)
  return pl.pallas_call(
      _msg_tc,
      grid=grid,
      in_specs=[
          pl.BlockSpec((_MSG_BLK, H), lambda g: (g, 0)),
          pl.BlockSpec((_MSG_BLK, H), lambda g: (g, 0)),
          pl.BlockSpec((H, H), lambda g: (0, 0)),
          pl.BlockSpec((1, H), lambda g: (0, 0)),
      ],
      out_specs=pl.BlockSpec((_MSG_BLK, H), lambda g: (g, 0)),
      out_shape=jax.ShapeDtypeStruct((E, H), jnp.float32),
  )(gi, gj, w2, b2.reshape(1, H))


def _final_tc(s_ref, dp_ref, xres_ref, verts_ref, wo1_ref, bo1_ref,
              wo2_ref, bo2_ref, vp_ref, dv_ref):
  d = dp_ref[0, :, 0:1] + dp_ref[1, :, 0:1]
  rdeg = 1.0 / jnp.maximum(d, 1.0)
  x3 = xres_ref[...] + (s_ref[0] + s_ref[1]) * rdeg
  h = jnp.maximum(jnp.dot(x3, wo1_ref[...], preferred_element_type=jnp.float32)
                  + bo1_ref[...], 0.0)
  dv = jnp.dot(h, wo2_ref[...], preferred_element_type=jnp.float32) + bo2_ref[...]
  dv_ref[...] = dv
  vp_ref[...] = verts_ref[...] + dv


def _final(s_part, deg_part, x_res, verts, wo1, bo1, wo2, bo2):
  return pl.pallas_call(
      _final_tc,
      out_shape=(jax.ShapeDtypeStruct((N, 3), jnp.float32),
                 jax.ShapeDtypeStruct((N, 3), jnp.float32)),
  )(s_part, deg_part, x_res, verts, wo1, bo1.reshape(1, H), wo2,
    bo2.reshape(1, 3))


def kernel(verts, edge_index, W1_0, b1_0, W2_0, b2_0, W1_1, b1_1, W2_1, b2_1,
           W1_2, b1_2, W2_2, b2_2, Wo1, bo1, Wo2, bo2):
  idx_i = edge_index[0].astype(jnp.int32).reshape(1, E)
  idx_j = edge_index[1].astype(jnp.int32).reshape(1, E)
  zeros_nh = jnp.zeros((N, H), jnp.float32)
  ones_wh = jnp.ones((WIN, H), jnp.float32)

  deg_part = _sc_degree(idx_j, zeros_nh, ones_wh)

  p, q = _pq0(verts, W1_0, b1_0)
  gi, gj = _sc_gather(p, q, idx_i, idx_j)
  m = _messages(gi, gj, W2_0, b2_0)
  s_part = _sc_scatter_add(m, idx_j, zeros_nh)

  x1, p, q = _combine_pq(s_part, deg_part, None, W1_1, b1_1)
  gi, gj = _sc_gather(p, q, idx_i, idx_j)
  m = _messages(gi, gj, W2_1, b2_1)
  s_part = _sc_scatter_add(m, idx_j, zeros_nh)

  x2, p, q = _combine_pq(s_part, deg_part, x1, W1_2, b1_2)
  gi, gj = _sc_gather(p, q, idx_i, idx_j)
  m = _messages(gi, gj, W2_2, b2_2)
  s_part = _sc_scatter_add(m, idx_j, zeros_nh)

  return _final(s_part, deg_part, x2, verts, Wo1, bo1, Wo2, bo2)
